# baseline (speedup 1.0000x reference)
import functools

import jax
import jax.numpy as jnp
from jax import lax
from jax.experimental import pallas as pl
from jax.experimental.pallas import tpu as pltpu
from jax.experimental.pallas import tpu_sc as plsc

_CH = 80


def _make_edge_agg(n_ent_pad, d_in, chunks_per_tile):
  mesh = plsc.VectorSubcoreMesh(core_axis_name="c", subcore_axis_name="s")
  ns = 16
  rows_per_tile = n_ent_pad // ns
  nlane = d_in // 16
  nchunks = chunks_per_tile

  @functools.partial(
      pl.kernel,
      mesh=mesh,
      out_type=[
          jax.ShapeDtypeStruct((n_ent_pad, d_in), jnp.float32),
          jax.ShapeDtypeStruct((n_ent_pad, d_in), jnp.float32),
      ],
      scratch_types=[
          pltpu.VMEM_SHARED((n_ent_pad, d_in), jnp.float32),
          pltpu.VMEM((2, _CH), jnp.int32),
          pltpu.VMEM((2, _CH), jnp.int32),
          pltpu.VMEM((3, _CH), jnp.int32),
          pltpu.VMEM((2, _CH), jnp.float32),
          pltpu.VMEM((2, _CH, d_in), jnp.float32),
          pltpu.VMEM((2, _CH, d_in), jnp.float32),
          pltpu.SemaphoreType.DMA,
          pltpu.SemaphoreType.DMA,
          pltpu.SemaphoreType.DMA,
          pltpu.SemaphoreType.DMA,
      ],
  )
  def edge_agg(ent_hbm, rel_hbm, src_hbm, typ_hbm, dst_hbm, nrm_hbm, zero_hbm,
               out_in, out_out, shared, src_v, typ_v, dst_v, nrm_v,
               erow, rrow, sem_a, sem_b, sem_i, sem_s):
    c = lax.axis_index("c")
    s = lax.axis_index("s")
    rb = s * rows_per_tile
    pltpu.sync_copy(zero_hbm.at[pl.ds(rb, rows_per_tile)],
                    shared.at[pl.ds(rb, rows_per_tile)])
    start = (c * ns + s) * nchunks
    plsc.subcore_barrier()

    def fire_idx(i, slot, dslot):
      base = (start + i) * _CH
      pltpu.async_copy(src_hbm.at[pl.ds(base, _CH)], src_v.at[slot], sem_i)
      pltpu.async_copy(typ_hbm.at[pl.ds(base, _CH)], typ_v.at[slot], sem_i)
      pltpu.async_copy(dst_hbm.at[pl.ds(base, _CH)], dst_v.at[dslot], sem_i)
      pltpu.async_copy(nrm_hbm.at[pl.ds(base, _CH)], nrm_v.at[slot], sem_i)

    def wait_idx(slot, dslot):
      z = pl.ds(0, _CH)
      pltpu.make_async_copy(src_hbm.at[z], src_v.at[slot], sem_i).wait()
      pltpu.make_async_copy(typ_hbm.at[z], typ_v.at[slot], sem_i).wait()
      pltpu.make_async_copy(dst_hbm.at[z], dst_v.at[dslot], sem_i).wait()
      pltpu.make_async_copy(nrm_hbm.at[z], nrm_v.at[slot], sem_i).wait()

    def wait_scatter(dslot):
      pltpu.make_async_copy(erow.at[0], shared.at[dst_v.at[dslot]],
                            sem_s).wait()

    def fire_rows(slot):
      pltpu.async_copy(ent_hbm.at[src_v.at[slot]], erow.at[slot], sem_a)
      pltpu.async_copy(rel_hbm.at[typ_v.at[slot]], rrow.at[slot], sem_b)

    def drain_rows(slot):
      pltpu.make_async_copy(ent_hbm.at[src_v.at[0]], erow.at[slot], sem_a).wait()
      pltpu.make_async_copy(rel_hbm.at[typ_v.at[0]], rrow.at[slot], sem_b).wait()

    fire_idx(0, 0, 0)
    wait_idx(0, 0)
    fire_rows(0)
    fire_idx(1, 1, 1)

    dnums = lax.GatherDimensionNumbers(
        offset_dims=(), collapsed_slice_dims=(0,), start_index_map=(0,))

    def chunk_body(i, carry):
      slot = lax.rem(i, 2)
      nxt = lax.rem(i + 1, 2)
      dslot = lax.rem(i, 3)

      drain_rows(slot)

      @pl.when(i >= 1)
      def _():
        wait_scatter(lax.rem(i + 2, 3))

      @pl.when(i + 1 < nchunks)
      def _():
        wait_idx(nxt, lax.rem(i + 1, 3))
        fire_rows(nxt)

      def group_body(g, carry2):
        gv = nrm_v[slot, pl.ds(g * 16, 16)]
        for lane in range(16):
          lidx = jnp.full((16, 1), lane, jnp.int32)
          n = lax.gather(gv, lidx, dnums, slice_sizes=(1,),
                         mode=lax.GatherScatterMode.PROMISE_IN_BOUNDS)
          e = g * 16 + lane
          for j in range(nlane):
            sl = pl.ds(j * 16, 16)
            erow[slot, e, sl] = (erow[slot, e, sl] - rrow[slot, e, sl]) * n
        return carry2

      lax.fori_loop(0, _CH // 16, group_body, 0)
      pltpu.async_copy(erow.at[slot], shared.at[dst_v.at[dslot]], sem_s,
                       add=True)

      @pl.when(i + 2 < nchunks)
      def _():
        fire_idx(i + 2, slot, lax.rem(i + 2, 3))

      return carry

    lax.fori_loop(0, nchunks, chunk_body, 0)
    wait_scatter(lax.rem(nchunks - 1, 3))
    plsc.subcore_barrier()

    @pl.when(c == 0)
    def _():
      pltpu.sync_copy(shared.at[pl.ds(rb, rows_per_tile)],
                      out_in.at[pl.ds(rb, rows_per_tile)])

    @pl.when(c == 1)
    def _():
      pltpu.sync_copy(shared.at[pl.ds(rb, rows_per_tile)],
                      out_out.at[pl.ds(rb, rows_per_tile)])

  return edge_agg


def _make_pair_gather(d, batch):
  mesh = plsc.VectorSubcoreMesh(core_axis_name="c", subcore_axis_name="s")
  nw = 32
  per = batch // nw

  @functools.partial(
      pl.kernel,
      mesh=mesh,
      out_type=[
          jax.ShapeDtypeStruct((batch, d), jnp.float32),
          jax.ShapeDtypeStruct((batch, d), jnp.float32),
      ],
      scratch_types=[
          pltpu.VMEM((per,), jnp.int32),
          pltpu.VMEM((per,), jnp.int32),
          pltpu.VMEM((per, d), jnp.float32),
          pltpu.VMEM((per, d), jnp.float32),
          pltpu.SemaphoreType.DMA,
          pltpu.SemaphoreType.DMA,
      ],
  )
  def pair_gather(x_hbm, r_hbm, head_hbm, rela_hbm, out_x, out_r,
                  hidx, ridx, xrow, rrow, sem_a, sem_b):
    c = lax.axis_index("c")
    s = lax.axis_index("s")
    base = (s * 2 + c) * per
    pltpu.sync_copy(head_hbm.at[pl.ds(base, per)], hidx)
    pltpu.sync_copy(rela_hbm.at[pl.ds(base, per)], ridx)
    ga = pltpu.async_copy(x_hbm.at[hidx], xrow, sem_a)
    gb = pltpu.async_copy(r_hbm.at[ridx], rrow, sem_b)
    ga.wait()
    gb.wait()
    pltpu.sync_copy(xrow, out_x.at[pl.ds(base, per)])
    pltpu.sync_copy(rrow, out_r.at[pl.ds(base, per)])

  return pair_gather


def _xpre_body(agg_in_ref, agg_out_ref, ent_ref, in_w_ref, out_w_ref,
               loop_w_ref, loop_rel_ref, bias_ref, rel_ref, w_rel_ref,
               xpre_ref, stats_ref, r_ref, acc, *, nblk, rblk, n_ent):
  i = pl.program_id(0)
  f32 = jnp.float32
  xp = jnp.dot(agg_in_ref[...], in_w_ref[...], preferred_element_type=f32)
  xp += jnp.dot(agg_out_ref[...], out_w_ref[...], preferred_element_type=f32)
  xp += jnp.dot(ent_ref[...] - loop_rel_ref[...], loop_w_ref[...],
                preferred_element_type=f32)
  xp = xp * (1.0 / 3.0) + bias_ref[...]
  xpre_ref[...] = xp

  @pl.when(i == 0)
  def _():
    acc[...] = jnp.zeros_like(acc)
    r_ref[...] = jnp.dot(rel_ref[...], w_rel_ref[...], preferred_element_type=f32)

  row = i * rblk + lax.broadcasted_iota(jnp.int32, xp.shape, 0)
  xpm = jnp.where(row < n_ent, xp, 0.0)
  acc[0:1, :] += jnp.sum(xpm, axis=0, keepdims=True)
  acc[1:2, :] += jnp.sum(xpm * xpm, axis=0, keepdims=True)

  @pl.when(i == nblk - 1)
  def _():
    stats_ref[...] = acc[...]


def _decoder_body(xh_ref, rh_ref, stats_ref, gamma_ref, beta_ref,
                  emb_ref, bent_ref, out_ref, obj, *, n_ent, bblk):
  i = pl.program_id(0)

  @pl.when(i == 0)
  def _():
    inv_n = 1.0 / n_ent
    mean = stats_ref[0:1, :] * inv_n
    var = stats_ref[1:2, :] * inv_n - mean * mean
    xn = (xh_ref[...] - mean) * lax.rsqrt(var + 1e-5)
    xn = jnp.tanh(xn * gamma_ref[...] + beta_ref[...])
    obj[...] = xn * rh_ref[...]

  logits = lax.dot_general(obj[pl.ds(i * bblk, bblk), :], emb_ref[...],
                           (((1,), (1,)), ((), ())),
                           preferred_element_type=jnp.float32)
  logits += bent_ref[...]
  out_ref[...] = jax.nn.sigmoid(logits)


def kernel(ent_emb, rel_emb, in_w, out_w, loop_w, w_rel, loop_rel, bias_cov,
           bn_gamma, bn_beta, b_ent, emb_ent, edge_index, edge_type, edge_norm,
           triples):
  n_ent, d_in = ent_emb.shape
  d_out = in_w.shape[1]
  n_rel = rel_emb.shape[0]
  n_edges = edge_norm.shape[0]
  batch = triples.shape[0]
  chunks = n_edges // _CH
  cpc = chunks // 2
  ns = 16
  cpt = -(-cpc // ns)
  cpt = -(-cpt // 8) * 8
  cpc_pad = cpt * ns

  half = n_edges // 2
  npad = (cpc_pad - cpc) * _CH

  def _chunked(a, fill):
    zpad = jnp.full((npad,), fill, a.dtype)
    return jnp.concatenate([a[:half], zpad, a[half:], zpad])

  src_c = _chunked(edge_index[0], 0)
  dst_c = _chunked(edge_index[1], 0)
  typ_c = _chunked(edge_type, 0)
  nrm_c = _chunked(edge_norm, 0.0)
  n_ent_pad = -(-n_ent // (80 * ns)) * (80 * ns)
  zeros = jnp.zeros((n_ent_pad, d_in), jnp.float32)

  edge_agg = _make_edge_agg(n_ent_pad, d_in, cpt)
  agg_in, agg_out = edge_agg(ent_emb, rel_emb, src_c, typ_c, dst_c, nrm_c,
                             zeros)

  ent_p = jnp.concatenate(
      [ent_emb, jnp.zeros((n_ent_pad - n_ent, d_in), jnp.float32)], axis=0)
  rblk = 1024
  nblk = n_ent_pad // rblk
  xpre, stats, r = pl.pallas_call(
      functools.partial(_xpre_body, nblk=nblk, rblk=rblk, n_ent=n_ent),
      grid=(nblk,),
      in_specs=[
          pl.BlockSpec((rblk, d_in), lambda i: (i, 0)),
          pl.BlockSpec((rblk, d_in), lambda i: (i, 0)),
          pl.BlockSpec((rblk, d_in), lambda i: (i, 0)),
          pl.BlockSpec((d_in, d_out), lambda i: (0, 0)),
          pl.BlockSpec((d_in, d_out), lambda i: (0, 0)),
          pl.BlockSpec((d_in, d_out), lambda i: (0, 0)),
          pl.BlockSpec((1, d_in), lambda i: (0, 0)),
          pl.BlockSpec((1, d_out), lambda i: (0, 0)),
          pl.BlockSpec((n_rel, d_in), lambda i: (0, 0)),
          pl.BlockSpec((d_in, d_out), lambda i: (0, 0)),
      ],
      out_specs=[
          pl.BlockSpec((rblk, d_out), lambda i: (i, 0)),
          pl.BlockSpec((8, d_out), lambda i: (0, 0)),
          pl.BlockSpec((n_rel, d_out), lambda i: (0, 0)),
      ],
      out_shape=[
          jax.ShapeDtypeStruct((n_ent_pad, d_out), jnp.float32),
          jax.ShapeDtypeStruct((8, d_out), jnp.float32),
          jax.ShapeDtypeStruct((n_rel, d_out), jnp.float32),
      ],
      scratch_shapes=[pltpu.VMEM((8, d_out), jnp.float32)],
  )(agg_in, agg_out, ent_p, in_w, out_w, loop_w,
    loop_rel, bias_cov.reshape(1, d_out), rel_emb, w_rel)

  pair_gather = _make_pair_gather(d_out, batch)
  head = jnp.asarray(triples[:, 0], jnp.int32)
  rela = jnp.asarray(triples[:, 1], jnp.int32)
  xh, rh = pair_gather(xpre, r, head, rela)

  bblk = 128
  nbb = batch // bblk
  score = pl.pallas_call(
      functools.partial(_decoder_body, n_ent=float(n_ent), bblk=bblk),
      grid=(nbb,),
      in_specs=[
          pl.BlockSpec((batch, d_out), lambda i: (0, 0)),
          pl.BlockSpec((batch, d_out), lambda i: (0, 0)),
          pl.BlockSpec((8, d_out), lambda i: (0, 0)),
          pl.BlockSpec((1, d_out), lambda i: (0, 0)),
          pl.BlockSpec((1, d_out), lambda i: (0, 0)),
          pl.BlockSpec((n_ent, d_out), lambda i: (0, 0)),
          pl.BlockSpec((1, n_ent), lambda i: (0, 0)),
      ],
      out_specs=pl.BlockSpec((bblk, n_ent), lambda i: (i, 0)),
      out_shape=jax.ShapeDtypeStruct((batch, n_ent), jnp.float32),
      scratch_shapes=[pltpu.VMEM((batch, d_out), jnp.float32)],
  )(xh, rh, stats, bn_gamma.reshape(1, d_out), bn_beta.reshape(1, d_out),
    emb_ent, b_ent.reshape(1, n_ent))

  return score


# speedup vs baseline: 1.3325x; 1.3325x over previous
import functools

import jax
import jax.numpy as jnp
from jax import lax
from jax.experimental import pallas as pl
from jax.experimental.pallas import tpu as pltpu
from jax.experimental.pallas import tpu_sc as plsc

_CH = 64


def _make_edge_agg(n_ent_pad, d_in, chunks_per_tile):
  mesh = plsc.VectorSubcoreMesh(core_axis_name="c", subcore_axis_name="s")
  ns = 16
  rows_per_tile = n_ent_pad // ns
  nlane = d_in // 16
  nchunks = chunks_per_tile

  @functools.partial(
      pl.kernel,
      mesh=mesh,
      out_type=[
          jax.ShapeDtypeStruct((n_ent_pad, d_in), jnp.float32),
          jax.ShapeDtypeStruct((n_ent_pad, d_in), jnp.float32),
      ],
      scratch_types=[
          pltpu.VMEM_SHARED((n_ent_pad, d_in), jnp.float32),
          pltpu.VMEM((2, _CH), jnp.int32),
          pltpu.VMEM((2, _CH), jnp.int32),
          pltpu.VMEM((3, _CH), jnp.int32),
          pltpu.VMEM((2, _CH), jnp.float32),
          pltpu.VMEM((2, _CH, d_in), jnp.float32),
          pltpu.VMEM((2, _CH, d_in), jnp.float32),
          pltpu.SemaphoreType.DMA,
          pltpu.SemaphoreType.DMA,
          pltpu.SemaphoreType.DMA,
          pltpu.SemaphoreType.DMA,
      ],
  )
  def edge_agg(ent_hbm, rel_hbm, src_hbm, typ_hbm, dst_hbm, nrm_hbm, zero_hbm,
               out_in, out_out, shared, src_v, typ_v, dst_v, nrm_v,
               erow, rrow, sem_a, sem_b, sem_i, sem_s):
    c = lax.axis_index("c")
    s = lax.axis_index("s")
    rb = s * rows_per_tile
    pltpu.sync_copy(zero_hbm.at[pl.ds(rb, rows_per_tile)],
                    shared.at[pl.ds(rb, rows_per_tile)])
    start = (c * ns + s) * nchunks
    plsc.subcore_barrier()

    def fire_idx(i, slot, dslot):
      base = (start + i) * _CH
      pltpu.async_copy(src_hbm.at[pl.ds(base, _CH)], src_v.at[slot], sem_i)
      pltpu.async_copy(typ_hbm.at[pl.ds(base, _CH)], typ_v.at[slot], sem_i)
      pltpu.async_copy(dst_hbm.at[pl.ds(base, _CH)], dst_v.at[dslot], sem_i)
      pltpu.async_copy(nrm_hbm.at[pl.ds(base, _CH)], nrm_v.at[slot], sem_i)

    def wait_idx(slot, dslot):
      z = pl.ds(0, _CH)
      pltpu.make_async_copy(src_hbm.at[z], src_v.at[slot], sem_i).wait()
      pltpu.make_async_copy(typ_hbm.at[z], typ_v.at[slot], sem_i).wait()
      pltpu.make_async_copy(dst_hbm.at[z], dst_v.at[dslot], sem_i).wait()
      pltpu.make_async_copy(nrm_hbm.at[z], nrm_v.at[slot], sem_i).wait()

    def wait_scatter(dslot):
      pltpu.make_async_copy(erow.at[0], shared.at[dst_v.at[dslot]],
                            sem_s).wait()

    def fire_rows(slot):
      pltpu.async_copy(ent_hbm.at[src_v.at[slot]], erow.at[slot], sem_a)
      pltpu.async_copy(rel_hbm.at[typ_v.at[slot]], rrow.at[slot], sem_b)

    def drain_rows(slot):
      pltpu.make_async_copy(ent_hbm.at[src_v.at[0]], erow.at[slot], sem_a).wait()
      pltpu.make_async_copy(rel_hbm.at[typ_v.at[0]], rrow.at[slot], sem_b).wait()

    fire_idx(0, 0, 0)
    wait_idx(0, 0)
    fire_rows(0)
    fire_idx(1, 1, 1)

    dnums = lax.GatherDimensionNumbers(
        offset_dims=(), collapsed_slice_dims=(0,), start_index_map=(0,))

    def chunk_body(i, carry):
      slot = lax.rem(i, 2)
      nxt = lax.rem(i + 1, 2)
      dslot = lax.rem(i, 3)

      drain_rows(slot)

      @pl.when(i >= 1)
      def _():
        wait_scatter(lax.rem(i + 2, 3))

      @pl.when(i + 1 < nchunks)
      def _():
        wait_idx(nxt, lax.rem(i + 1, 3))
        fire_rows(nxt)

      def group_body(g, carry2):
        gv = nrm_v[slot, pl.ds(g * 16, 16)]
        for lane in range(16):
          lidx = jnp.full((16, 1), lane, jnp.int32)
          n = lax.gather(gv, lidx, dnums, slice_sizes=(1,),
                         mode=lax.GatherScatterMode.PROMISE_IN_BOUNDS)
          e = g * 16 + lane
          for j in range(nlane):
            sl = pl.ds(j * 16, 16)
            erow[slot, e, sl] = (erow[slot, e, sl] - rrow[slot, e, sl]) * n
        return carry2

      lax.fori_loop(0, _CH // 16, group_body, 0)
      pltpu.async_copy(erow.at[slot], shared.at[dst_v.at[dslot]], sem_s,
                       add=True)

      @pl.when(i + 2 < nchunks)
      def _():
        fire_idx(i + 2, slot, lax.rem(i + 2, 3))

      return carry

    lax.fori_loop(0, nchunks, chunk_body, 0)
    wait_scatter(lax.rem(nchunks - 1, 3))
    plsc.subcore_barrier()

    @pl.when(c == 0)
    def _():
      pltpu.sync_copy(shared.at[pl.ds(rb, rows_per_tile)],
                      out_in.at[pl.ds(rb, rows_per_tile)])

    @pl.when(c == 1)
    def _():
      pltpu.sync_copy(shared.at[pl.ds(rb, rows_per_tile)],
                      out_out.at[pl.ds(rb, rows_per_tile)])

  return edge_agg


def _make_pair_gather(d, batch):
  mesh = plsc.VectorSubcoreMesh(core_axis_name="c", subcore_axis_name="s")
  nw = 32
  per = batch // nw

  @functools.partial(
      pl.kernel,
      mesh=mesh,
      out_type=[
          jax.ShapeDtypeStruct((batch, d), jnp.float32),
          jax.ShapeDtypeStruct((batch, d), jnp.float32),
      ],
      scratch_types=[
          pltpu.VMEM((per,), jnp.int32),
          pltpu.VMEM((per,), jnp.int32),
          pltpu.VMEM((per, d), jnp.float32),
          pltpu.VMEM((per, d), jnp.float32),
          pltpu.SemaphoreType.DMA,
          pltpu.SemaphoreType.DMA,
      ],
  )
  def pair_gather(x_hbm, r_hbm, head_hbm, rela_hbm, out_x, out_r,
                  hidx, ridx, xrow, rrow, sem_a, sem_b):
    c = lax.axis_index("c")
    s = lax.axis_index("s")
    base = (s * 2 + c) * per
    pltpu.sync_copy(head_hbm.at[pl.ds(base, per)], hidx)
    pltpu.sync_copy(rela_hbm.at[pl.ds(base, per)], ridx)
    ga = pltpu.async_copy(x_hbm.at[hidx], xrow, sem_a)
    gb = pltpu.async_copy(r_hbm.at[ridx], rrow, sem_b)
    ga.wait()
    gb.wait()
    pltpu.sync_copy(xrow, out_x.at[pl.ds(base, per)])
    pltpu.sync_copy(rrow, out_r.at[pl.ds(base, per)])

  return pair_gather


def _xpre_body(agg_in_ref, agg_out_ref, ent_ref, in_w_ref, out_w_ref,
               loop_w_ref, loop_rel_ref, bias_ref, rel_ref, w_rel_ref,
               xpre_ref, stats_ref, r_ref, acc, *, nblk, rblk, n_ent):
  i = pl.program_id(0)
  f32 = jnp.float32
  xp = jnp.dot(agg_in_ref[...], in_w_ref[...], preferred_element_type=f32)
  xp += jnp.dot(agg_out_ref[...], out_w_ref[...], preferred_element_type=f32)
  xp += jnp.dot(ent_ref[...] - loop_rel_ref[...], loop_w_ref[...],
                preferred_element_type=f32)
  xp = xp * (1.0 / 3.0) + bias_ref[...]
  xpre_ref[...] = xp

  @pl.when(i == 0)
  def _():
    acc[...] = jnp.zeros_like(acc)
    r_ref[...] = jnp.dot(rel_ref[...], w_rel_ref[...], preferred_element_type=f32)

  row = i * rblk + lax.broadcasted_iota(jnp.int32, xp.shape, 0)
  xpm = jnp.where(row < n_ent, xp, 0.0)
  acc[0:1, :] += jnp.sum(xpm, axis=0, keepdims=True)
  acc[1:2, :] += jnp.sum(xpm * xpm, axis=0, keepdims=True)

  @pl.when(i == nblk - 1)
  def _():
    stats_ref[...] = acc[...]


def _decoder_body(xh_ref, rh_ref, stats_ref, gamma_ref, beta_ref,
                  emb_ref, bent_ref, out_ref, obj, *, n_ent, bblk):
  i = pl.program_id(0)

  @pl.when(i == 0)
  def _():
    inv_n = 1.0 / n_ent
    mean = stats_ref[0:1, :] * inv_n
    var = stats_ref[1:2, :] * inv_n - mean * mean
    xn = (xh_ref[...] - mean) * lax.rsqrt(var + 1e-5)
    xn = jnp.tanh(xn * gamma_ref[...] + beta_ref[...])
    obj[...] = xn * rh_ref[...]

  logits = lax.dot_general(obj[pl.ds(i * bblk, bblk), :], emb_ref[...],
                           (((1,), (1,)), ((), ())),
                           preferred_element_type=jnp.float32)
  logits += bent_ref[...]
  out_ref[...] = jax.nn.sigmoid(logits)


def kernel(ent_emb, rel_emb, in_w, out_w, loop_w, w_rel, loop_rel, bias_cov,
           bn_gamma, bn_beta, b_ent, emb_ent, edge_index, edge_type, edge_norm,
           triples):
  n_ent, d_in = ent_emb.shape
  d_out = in_w.shape[1]
  n_rel = rel_emb.shape[0]
  n_edges = edge_norm.shape[0]
  batch = triples.shape[0]
  chunks = n_edges // _CH
  cpc = chunks // 2
  ns = 16
  cpt = -(-cpc // ns)
  cpt = -(-cpt // 8) * 8
  cpc_pad = cpt * ns

  half = n_edges // 2
  npad = (cpc_pad - cpc) * _CH

  def _chunked(a, fill):
    zpad = jnp.full((npad,), fill, a.dtype)
    return jnp.concatenate([a[:half], zpad, a[half:], zpad])

  src_c = _chunked(edge_index[0], 0)
  dst_c = _chunked(edge_index[1], 0)
  typ_c = _chunked(edge_type, 0)
  nrm_c = _chunked(edge_norm, 0.0)
  n_ent_pad = -(-n_ent // (80 * ns)) * (80 * ns)
  zeros = jnp.zeros((n_ent_pad, d_in), jnp.float32)

  edge_agg = _make_edge_agg(n_ent_pad, d_in, cpt)
  agg_in, agg_out = edge_agg(ent_emb, rel_emb, src_c, typ_c, dst_c, nrm_c,
                             zeros)

  ent_p = jnp.concatenate(
      [ent_emb, jnp.zeros((n_ent_pad - n_ent, d_in), jnp.float32)], axis=0)
  rblk = 1024
  nblk = n_ent_pad // rblk
  xpre, stats, r = pl.pallas_call(
      functools.partial(_xpre_body, nblk=nblk, rblk=rblk, n_ent=n_ent),
      grid=(nblk,),
      in_specs=[
          pl.BlockSpec((rblk, d_in), lambda i: (i, 0)),
          pl.BlockSpec((rblk, d_in), lambda i: (i, 0)),
          pl.BlockSpec((rblk, d_in), lambda i: (i, 0)),
          pl.BlockSpec((d_in, d_out), lambda i: (0, 0)),
          pl.BlockSpec((d_in, d_out), lambda i: (0, 0)),
          pl.BlockSpec((d_in, d_out), lambda i: (0, 0)),
          pl.BlockSpec((1, d_in), lambda i: (0, 0)),
          pl.BlockSpec((1, d_out), lambda i: (0, 0)),
          pl.BlockSpec((n_rel, d_in), lambda i: (0, 0)),
          pl.BlockSpec((d_in, d_out), lambda i: (0, 0)),
      ],
      out_specs=[
          pl.BlockSpec((rblk, d_out), lambda i: (i, 0)),
          pl.BlockSpec((8, d_out), lambda i: (0, 0)),
          pl.BlockSpec((n_rel, d_out), lambda i: (0, 0)),
      ],
      out_shape=[
          jax.ShapeDtypeStruct((n_ent_pad, d_out), jnp.float32),
          jax.ShapeDtypeStruct((8, d_out), jnp.float32),
          jax.ShapeDtypeStruct((n_rel, d_out), jnp.float32),
      ],
      scratch_shapes=[pltpu.VMEM((8, d_out), jnp.float32)],
  )(agg_in, agg_out, ent_p, in_w, out_w, loop_w,
    loop_rel, bias_cov.reshape(1, d_out), rel_emb, w_rel)

  pair_gather = _make_pair_gather(d_out, batch)
  head = jnp.asarray(triples[:, 0], jnp.int32)
  rela = jnp.asarray(triples[:, 1], jnp.int32)
  xh, rh = pair_gather(xpre, r, head, rela)

  bblk = 128
  nbb = batch // bblk
  score = pl.pallas_call(
      functools.partial(_decoder_body, n_ent=float(n_ent), bblk=bblk),
      grid=(nbb,),
      in_specs=[
          pl.BlockSpec((batch, d_out), lambda i: (0, 0)),
          pl.BlockSpec((batch, d_out), lambda i: (0, 0)),
          pl.BlockSpec((8, d_out), lambda i: (0, 0)),
          pl.BlockSpec((1, d_out), lambda i: (0, 0)),
          pl.BlockSpec((1, d_out), lambda i: (0, 0)),
          pl.BlockSpec((n_ent, d_out), lambda i: (0, 0)),
          pl.BlockSpec((1, n_ent), lambda i: (0, 0)),
      ],
      out_specs=pl.BlockSpec((bblk, n_ent), lambda i: (i, 0)),
      out_shape=jax.ShapeDtypeStruct((batch, n_ent), jnp.float32),
      scratch_shapes=[pltpu.VMEM((batch, d_out), jnp.float32)],
  )(xh, rh, stats, bn_gamma.reshape(1, d_out), bn_beta.reshape(1, d_out),
    emb_ent, b_ent.reshape(1, n_ent))

  return score


# speedup vs baseline: 1.3402x; 1.0058x over previous
import functools

import jax
import jax.numpy as jnp
from jax import lax
from jax.experimental import pallas as pl
from jax.experimental.pallas import tpu as pltpu
from jax.experimental.pallas import tpu_sc as plsc

_CH = 64


def _make_edge_agg(n_ent_pad, d_in, chunks_per_tile, batch):
  mesh = plsc.VectorSubcoreMesh(core_axis_name="c", subcore_axis_name="s")
  ns = 16
  rows_per_tile = n_ent_pad // ns
  nlane = d_in // 16
  nchunks = chunks_per_tile
  hpt = batch // ns

  @functools.partial(
      pl.kernel,
      mesh=mesh,
      out_type=[
          jax.ShapeDtypeStruct((n_ent_pad, d_in), jnp.float32),
          jax.ShapeDtypeStruct((n_ent_pad, d_in), jnp.float32),
          jax.ShapeDtypeStruct((batch, d_in), jnp.float32),
          jax.ShapeDtypeStruct((batch, d_in), jnp.float32),
          jax.ShapeDtypeStruct((batch, d_in), jnp.float32),
          jax.ShapeDtypeStruct((batch, d_in), jnp.float32),
      ],
      scratch_types=[
          pltpu.VMEM_SHARED((n_ent_pad, d_in), jnp.float32),
          pltpu.VMEM((2, _CH), jnp.int32),
          pltpu.VMEM((2, _CH), jnp.int32),
          pltpu.VMEM((3, _CH), jnp.int32),
          pltpu.VMEM((2, _CH), jnp.float32),
          pltpu.VMEM((2, _CH, d_in), jnp.float32),
          pltpu.VMEM((2, _CH, d_in), jnp.float32),
          pltpu.SemaphoreType.DMA,
          pltpu.SemaphoreType.DMA,
          pltpu.SemaphoreType.DMA,
          pltpu.SemaphoreType.DMA,
      ],
  )
  def edge_agg(ent_hbm, rel_hbm, src_hbm, typ_hbm, dst_hbm, nrm_hbm, zero_hbm,
               head_hbm, rela_hbm,
               out_in, out_out, out_ah, out_bh, out_ch, out_rg,
               shared, src_v, typ_v, dst_v, nrm_v,
               erow, rrow, sem_a, sem_b, sem_i, sem_s):
    c = lax.axis_index("c")
    s = lax.axis_index("s")
    rb = s * rows_per_tile
    pltpu.sync_copy(zero_hbm.at[pl.ds(rb, rows_per_tile)],
                    shared.at[pl.ds(rb, rows_per_tile)])
    start = (c * ns + s) * nchunks
    plsc.subcore_barrier()

    def fire_idx(i, slot, dslot):
      base = (start + i) * _CH
      pltpu.async_copy(src_hbm.at[pl.ds(base, _CH)], src_v.at[slot], sem_i)
      pltpu.async_copy(typ_hbm.at[pl.ds(base, _CH)], typ_v.at[slot], sem_i)
      pltpu.async_copy(dst_hbm.at[pl.ds(base, _CH)], dst_v.at[dslot], sem_i)
      pltpu.async_copy(nrm_hbm.at[pl.ds(base, _CH)], nrm_v.at[slot], sem_i)

    def wait_idx(slot, dslot):
      z = pl.ds(0, _CH)
      pltpu.make_async_copy(src_hbm.at[z], src_v.at[slot], sem_i).wait()
      pltpu.make_async_copy(typ_hbm.at[z], typ_v.at[slot], sem_i).wait()
      pltpu.make_async_copy(dst_hbm.at[z], dst_v.at[dslot], sem_i).wait()
      pltpu.make_async_copy(nrm_hbm.at[z], nrm_v.at[slot], sem_i).wait()

    def wait_scatter(dslot):
      pltpu.make_async_copy(erow.at[0], shared.at[dst_v.at[dslot]],
                            sem_s).wait()

    def fire_rows(slot):
      pltpu.async_copy(ent_hbm.at[src_v.at[slot]], erow.at[slot], sem_a)
      pltpu.async_copy(rel_hbm.at[typ_v.at[slot]], rrow.at[slot], sem_b)

    def drain_rows(slot):
      pltpu.make_async_copy(ent_hbm.at[src_v.at[0]], erow.at[slot], sem_a).wait()
      pltpu.make_async_copy(rel_hbm.at[typ_v.at[0]], rrow.at[slot], sem_b).wait()

    fire_idx(0, 0, 0)
    wait_idx(0, 0)
    fire_rows(0)
    fire_idx(1, 1, 1)

    dnums = lax.GatherDimensionNumbers(
        offset_dims=(), collapsed_slice_dims=(0,), start_index_map=(0,))

    def chunk_body(i, carry):
      slot = lax.rem(i, 2)
      nxt = lax.rem(i + 1, 2)
      dslot = lax.rem(i, 3)

      drain_rows(slot)

      @pl.when(i >= 1)
      def _():
        wait_scatter(lax.rem(i + 2, 3))

      @pl.when(i + 1 < nchunks)
      def _():
        wait_idx(nxt, lax.rem(i + 1, 3))
        fire_rows(nxt)

      def group_body(g, carry2):
        gv = nrm_v[slot, pl.ds(g * 16, 16)]
        for lane in range(16):
          lidx = jnp.full((16, 1), lane, jnp.int32)
          n = lax.gather(gv, lidx, dnums, slice_sizes=(1,),
                         mode=lax.GatherScatterMode.PROMISE_IN_BOUNDS)
          e = g * 16 + lane
          for j in range(nlane):
            sl = pl.ds(j * 16, 16)
            erow[slot, e, sl] = (erow[slot, e, sl] - rrow[slot, e, sl]) * n
        return carry2

      lax.fori_loop(0, _CH // 16, group_body, 0)
      pltpu.async_copy(erow.at[slot], shared.at[dst_v.at[dslot]], sem_s,
                       add=True)

      @pl.when(i + 2 < nchunks)
      def _():
        fire_idx(i + 2, slot, lax.rem(i + 2, 3))

      return carry

    lax.fori_loop(0, nchunks, chunk_body, 0)
    wait_scatter(lax.rem(nchunks - 1, 3))
    plsc.subcore_barrier()

    gb = s * hpt

    @pl.when(c == 0)
    def _():
      pltpu.sync_copy(shared.at[pl.ds(rb, rows_per_tile)],
                      out_in.at[pl.ds(rb, rows_per_tile)])
      pltpu.sync_copy(head_hbm.at[pl.ds(gb, hpt)], src_v.at[0])
      pltpu.sync_copy(shared.at[src_v.at[0]], erow.at[0])
      pltpu.sync_copy(ent_hbm.at[src_v.at[0]], rrow.at[0])
      pltpu.sync_copy(erow.at[0], out_ah.at[pl.ds(gb, hpt)])
      pltpu.sync_copy(rrow.at[0], out_ch.at[pl.ds(gb, hpt)])

    @pl.when(c == 1)
    def _():
      pltpu.sync_copy(shared.at[pl.ds(rb, rows_per_tile)],
                      out_out.at[pl.ds(rb, rows_per_tile)])
      pltpu.sync_copy(head_hbm.at[pl.ds(gb, hpt)], src_v.at[0])
      pltpu.sync_copy(rela_hbm.at[pl.ds(gb, hpt)], typ_v.at[0])
      pltpu.sync_copy(shared.at[src_v.at[0]], erow.at[0])
      pltpu.sync_copy(rel_hbm.at[typ_v.at[0]], rrow.at[0])
      pltpu.sync_copy(erow.at[0], out_bh.at[pl.ds(gb, hpt)])
      pltpu.sync_copy(rrow.at[0], out_rg.at[pl.ds(gb, hpt)])

  return edge_agg


def _stats_body(agg_in_ref, agg_out_ref, ent_ref, in_w_ref, out_w_ref,
                loop_w_ref, loop_rel_ref, bias_ref, stats_ref, acc,
                *, nblk, rblk, n_ent):
  i = pl.program_id(0)
  f32 = jnp.float32
  xp = jnp.dot(agg_in_ref[...], in_w_ref[...], preferred_element_type=f32)
  xp += jnp.dot(agg_out_ref[...], out_w_ref[...], preferred_element_type=f32)
  xp += jnp.dot(ent_ref[...] - loop_rel_ref[...], loop_w_ref[...],
                preferred_element_type=f32)
  xp = xp * (1.0 / 3.0) + bias_ref[...]

  @pl.when(i == 0)
  def _():
    acc[...] = jnp.zeros_like(acc)

  row = i * rblk + lax.broadcasted_iota(jnp.int32, xp.shape, 0)
  xpm = jnp.where(row < n_ent, xp, 0.0)
  acc[0:1, :] += jnp.sum(xpm, axis=0, keepdims=True)
  acc[1:2, :] += jnp.sum(xpm * xpm, axis=0, keepdims=True)

  @pl.when(i == nblk - 1)
  def _():
    stats_ref[...] = acc[...]


def _decoder_body(ah_ref, bh_ref, ch_ref, rg_ref, stats_ref, in_w_ref,
                  out_w_ref, loop_w_ref, w_rel_ref, loop_rel_ref, bias_ref,
                  gamma_ref, beta_ref, emb_ref, bent_ref, out_ref, obj,
                  *, n_ent, bblk):
  i = pl.program_id(0)

  @pl.when(i == 0)
  def _():
    f32 = jnp.float32
    xh = jnp.dot(ah_ref[...], in_w_ref[...], preferred_element_type=f32)
    xh += jnp.dot(bh_ref[...], out_w_ref[...], preferred_element_type=f32)
    xh += jnp.dot(ch_ref[...] - loop_rel_ref[...], loop_w_ref[...],
                  preferred_element_type=f32)
    xh = xh * (1.0 / 3.0) + bias_ref[...]
    rh = jnp.dot(rg_ref[...], w_rel_ref[...], preferred_element_type=f32)
    inv_n = 1.0 / n_ent
    mean = stats_ref[0:1, :] * inv_n
    var = stats_ref[1:2, :] * inv_n - mean * mean
    xn = (xh - mean) * lax.rsqrt(var + 1e-5)
    xn = jnp.tanh(xn * gamma_ref[...] + beta_ref[...])
    obj[...] = xn * rh

  logits = lax.dot_general(obj[pl.ds(i * bblk, bblk), :], emb_ref[...],
                           (((1,), (1,)), ((), ())),
                           preferred_element_type=jnp.float32)
  logits += bent_ref[...]
  out_ref[...] = jax.nn.sigmoid(logits)


def kernel(ent_emb, rel_emb, in_w, out_w, loop_w, w_rel, loop_rel, bias_cov,
           bn_gamma, bn_beta, b_ent, emb_ent, edge_index, edge_type, edge_norm,
           triples):
  n_ent, d_in = ent_emb.shape
  d_out = in_w.shape[1]
  n_rel = rel_emb.shape[0]
  n_edges = edge_norm.shape[0]
  batch = triples.shape[0]
  chunks = n_edges // _CH
  cpc = chunks // 2
  ns = 16
  cpt = -(-cpc // ns)
  cpt = -(-cpt // 8) * 8
  cpc_pad = cpt * ns

  half = n_edges // 2
  npad = (cpc_pad - cpc) * _CH

  def _chunked(a, fill):
    zpad = jnp.full((npad,), fill, a.dtype)
    return jnp.concatenate([a[:half], zpad, a[half:], zpad])

  src_c = _chunked(edge_index[0], 0)
  dst_c = _chunked(edge_index[1], 0)
  typ_c = _chunked(edge_type, 0)
  nrm_c = _chunked(edge_norm, 0.0)
  n_ent_pad = -(-n_ent // (80 * ns)) * (80 * ns)
  zeros = jnp.zeros((n_ent_pad, d_in), jnp.float32)

  head = jnp.asarray(triples[:, 0], jnp.int32)
  rela = jnp.asarray(triples[:, 1], jnp.int32)
  edge_agg = _make_edge_agg(n_ent_pad, d_in, cpt, batch)
  agg_in, agg_out, ah, bh, ch, rg = edge_agg(
      ent_emb, rel_emb, src_c, typ_c, dst_c, nrm_c, zeros, head, rela)

  ent_p = jnp.concatenate(
      [ent_emb, jnp.zeros((n_ent_pad - n_ent, d_in), jnp.float32)], axis=0)
  rblk = 1024
  nblk = n_ent_pad // rblk
  bias2 = bias_cov.reshape(1, d_out)
  stats = pl.pallas_call(
      functools.partial(_stats_body, nblk=nblk, rblk=rblk, n_ent=n_ent),
      grid=(nblk,),
      in_specs=[
          pl.BlockSpec((rblk, d_in), lambda i: (i, 0)),
          pl.BlockSpec((rblk, d_in), lambda i: (i, 0)),
          pl.BlockSpec((rblk, d_in), lambda i: (i, 0)),
          pl.BlockSpec((d_in, d_out), lambda i: (0, 0)),
          pl.BlockSpec((d_in, d_out), lambda i: (0, 0)),
          pl.BlockSpec((d_in, d_out), lambda i: (0, 0)),
          pl.BlockSpec((1, d_in), lambda i: (0, 0)),
          pl.BlockSpec((1, d_out), lambda i: (0, 0)),
      ],
      out_specs=pl.BlockSpec((8, d_out), lambda i: (0, 0)),
      out_shape=jax.ShapeDtypeStruct((8, d_out), jnp.float32),
      scratch_shapes=[pltpu.VMEM((8, d_out), jnp.float32)],
  )(agg_in, agg_out, ent_p, in_w, out_w, loop_w, loop_rel, bias2)

  bblk = 128
  nbb = batch // bblk
  score = pl.pallas_call(
      functools.partial(_decoder_body, n_ent=float(n_ent), bblk=bblk),
      grid=(nbb,),
      in_specs=[
          pl.BlockSpec((batch, d_in), lambda i: (0, 0)),
          pl.BlockSpec((batch, d_in), lambda i: (0, 0)),
          pl.BlockSpec((batch, d_in), lambda i: (0, 0)),
          pl.BlockSpec((batch, d_in), lambda i: (0, 0)),
          pl.BlockSpec((8, d_out), lambda i: (0, 0)),
          pl.BlockSpec((d_in, d_out), lambda i: (0, 0)),
          pl.BlockSpec((d_in, d_out), lambda i: (0, 0)),
          pl.BlockSpec((d_in, d_out), lambda i: (0, 0)),
          pl.BlockSpec((d_in, d_out), lambda i: (0, 0)),
          pl.BlockSpec((1, d_in), lambda i: (0, 0)),
          pl.BlockSpec((1, d_out), lambda i: (0, 0)),
          pl.BlockSpec((1, d_out), lambda i: (0, 0)),
          pl.BlockSpec((1, d_out), lambda i: (0, 0)),
          pl.BlockSpec((n_ent, d_out), lambda i: (0, 0)),
          pl.BlockSpec((1, n_ent), lambda i: (0, 0)),
      ],
      out_specs=pl.BlockSpec((bblk, n_ent), lambda i: (i, 0)),
      out_shape=jax.ShapeDtypeStruct((batch, n_ent), jnp.float32),
      scratch_shapes=[pltpu.VMEM((batch, d_out), jnp.float32)],
  )(ah, bh, ch, rg, stats, in_w, out_w, loop_w, w_rel, loop_rel, bias2,
    bn_gamma.reshape(1, d_out), bn_beta.reshape(1, d_out),
    emb_ent, b_ent.reshape(1, n_ent))

  return score


# speedup vs baseline: 1.3431x; 1.0022x over previous
import functools

import jax
import jax.numpy as jnp
from jax import lax
from jax.experimental import pallas as pl
from jax.experimental.pallas import tpu as pltpu
from jax.experimental.pallas import tpu_sc as plsc

_CH = 64


def _make_edge_agg(n_ent_pad, d_in, chunks_per_tile, batch):
  mesh = plsc.VectorSubcoreMesh(core_axis_name="c", subcore_axis_name="s")
  ns = 16
  rows_per_tile = n_ent_pad // ns
  nlane = d_in // 16
  nchunks = chunks_per_tile
  hpt = batch // ns

  @functools.partial(
      pl.kernel,
      mesh=mesh,
      out_type=[
          jax.ShapeDtypeStruct((n_ent_pad, d_in), jnp.float32),
          jax.ShapeDtypeStruct((n_ent_pad, d_in), jnp.float32),
          jax.ShapeDtypeStruct((batch, d_in), jnp.float32),
          jax.ShapeDtypeStruct((batch, d_in), jnp.float32),
          jax.ShapeDtypeStruct((batch, d_in), jnp.float32),
          jax.ShapeDtypeStruct((batch, d_in), jnp.float32),
      ],
      scratch_types=[
          pltpu.VMEM_SHARED((n_ent_pad, d_in), jnp.float32),
          pltpu.VMEM((2, _CH), jnp.int32),
          pltpu.VMEM((2, _CH), jnp.int32),
          pltpu.VMEM((3, _CH), jnp.int32),
          pltpu.VMEM((2, _CH), jnp.float32),
          pltpu.VMEM((2, _CH, d_in), jnp.float32),
          pltpu.VMEM((2, _CH, d_in), jnp.float32),
          pltpu.SemaphoreType.DMA,
          pltpu.SemaphoreType.DMA,
          pltpu.SemaphoreType.DMA,
          pltpu.SemaphoreType.DMA,
      ],
  )
  def edge_agg(ent_hbm, rel_hbm, src_hbm, typ_hbm, dst_hbm, nrm_hbm, zero_hbm,
               head_hbm, rela_hbm,
               out_in, out_out, out_ah, out_bh, out_ch, out_rg,
               shared, src_v, typ_v, dst_v, nrm_v,
               erow, rrow, sem_a, sem_b, sem_i, sem_s):
    c = lax.axis_index("c")
    s = lax.axis_index("s")
    rb = s * rows_per_tile
    pltpu.sync_copy(zero_hbm.at[pl.ds(rb, rows_per_tile)],
                    shared.at[pl.ds(rb, rows_per_tile)])
    start = (c * ns + s) * nchunks
    plsc.subcore_barrier()

    def fire_idx(i, slot, dslot):
      base = (start + i) * _CH
      pltpu.async_copy(src_hbm.at[pl.ds(base, _CH)], src_v.at[slot], sem_i)
      pltpu.async_copy(typ_hbm.at[pl.ds(base, _CH)], typ_v.at[slot], sem_i)
      pltpu.async_copy(dst_hbm.at[pl.ds(base, _CH)], dst_v.at[dslot], sem_i)
      pltpu.async_copy(nrm_hbm.at[pl.ds(base, _CH)], nrm_v.at[slot], sem_i)

    def wait_idx(slot, dslot):
      z = pl.ds(0, _CH)
      pltpu.make_async_copy(src_hbm.at[z], src_v.at[slot], sem_i).wait()
      pltpu.make_async_copy(typ_hbm.at[z], typ_v.at[slot], sem_i).wait()
      pltpu.make_async_copy(dst_hbm.at[z], dst_v.at[dslot], sem_i).wait()
      pltpu.make_async_copy(nrm_hbm.at[z], nrm_v.at[slot], sem_i).wait()

    def wait_scatter(dslot):
      pltpu.make_async_copy(erow.at[0], shared.at[dst_v.at[dslot]],
                            sem_s).wait()

    def fire_rows(slot):
      pltpu.async_copy(ent_hbm.at[src_v.at[slot]], erow.at[slot], sem_a)
      pltpu.async_copy(rel_hbm.at[typ_v.at[slot]], rrow.at[slot], sem_b)

    def drain_rows(slot):
      pltpu.make_async_copy(ent_hbm.at[src_v.at[0]], erow.at[slot], sem_a).wait()
      pltpu.make_async_copy(rel_hbm.at[typ_v.at[0]], rrow.at[slot], sem_b).wait()

    fire_idx(0, 0, 0)
    wait_idx(0, 0)
    fire_rows(0)
    fire_idx(1, 1, 1)

    dnums = lax.GatherDimensionNumbers(
        offset_dims=(), collapsed_slice_dims=(0,), start_index_map=(0,))

    def chunk_body(i, carry):
      slot = lax.rem(i, 2)
      nxt = lax.rem(i + 1, 2)
      dslot = lax.rem(i, 3)

      drain_rows(slot)

      @pl.when(i >= 1)
      def _():
        wait_scatter(lax.rem(i + 2, 3))

      @pl.when(i + 1 < nchunks)
      def _():
        wait_idx(nxt, lax.rem(i + 1, 3))
        fire_rows(nxt)

      def group_body(g, carry2):
        gv = nrm_v[slot, pl.ds(g * 16, 16)]
        for lane in range(16):
          lidx = jnp.full((16, 1), lane, jnp.int32)
          n = lax.gather(gv, lidx, dnums, slice_sizes=(1,),
                         mode=lax.GatherScatterMode.PROMISE_IN_BOUNDS)
          e = g * 16 + lane
          for j in range(nlane):
            sl = pl.ds(j * 16, 16)
            erow[slot, e, sl] = (erow[slot, e, sl] - rrow[slot, e, sl]) * n
        return carry2

      lax.fori_loop(0, _CH // 16, group_body, 0)
      pltpu.async_copy(erow.at[slot], shared.at[dst_v.at[dslot]], sem_s,
                       add=True)

      @pl.when(i + 2 < nchunks)
      def _():
        fire_idx(i + 2, slot, lax.rem(i + 2, 3))

      return carry

    lax.fori_loop(0, nchunks, chunk_body, 0)
    wait_scatter(lax.rem(nchunks - 1, 3))
    plsc.subcore_barrier()

    gb = s * hpt

    @pl.when(c == 0)
    def _():
      pltpu.sync_copy(shared.at[pl.ds(rb, rows_per_tile)],
                      out_in.at[pl.ds(rb, rows_per_tile)])
      pltpu.sync_copy(head_hbm.at[pl.ds(gb, hpt)], src_v.at[0])
      pltpu.sync_copy(shared.at[src_v.at[0]], erow.at[0])
      pltpu.sync_copy(ent_hbm.at[src_v.at[0]], rrow.at[0])
      pltpu.sync_copy(erow.at[0], out_ah.at[pl.ds(gb, hpt)])
      pltpu.sync_copy(rrow.at[0], out_ch.at[pl.ds(gb, hpt)])

    @pl.when(c == 1)
    def _():
      pltpu.sync_copy(shared.at[pl.ds(rb, rows_per_tile)],
                      out_out.at[pl.ds(rb, rows_per_tile)])
      pltpu.sync_copy(head_hbm.at[pl.ds(gb, hpt)], src_v.at[0])
      pltpu.sync_copy(rela_hbm.at[pl.ds(gb, hpt)], typ_v.at[0])
      pltpu.sync_copy(shared.at[src_v.at[0]], erow.at[0])
      pltpu.sync_copy(rel_hbm.at[typ_v.at[0]], rrow.at[0])
      pltpu.sync_copy(erow.at[0], out_bh.at[pl.ds(gb, hpt)])
      pltpu.sync_copy(rrow.at[0], out_rg.at[pl.ds(gb, hpt)])

  return edge_agg


def _fused_body(agg_in_ref, agg_out_ref, ent_ref, ah_ref, bh_ref, ch_ref,
                rg_ref, in_w_ref, out_w_ref, loop_w_ref, w_rel_ref,
                loop_rel_ref, bias_ref, gamma_ref, beta_ref, emb_ref,
                bent_ref, out_ref, acc, obj, *, nblk, rblk, n_ent, bblk):
  i = pl.program_id(0)
  f32 = jnp.float32

  @pl.when(i < nblk)
  def _():
    xp = jnp.dot(agg_in_ref[...], in_w_ref[...], preferred_element_type=f32)
    xp += jnp.dot(agg_out_ref[...], out_w_ref[...], preferred_element_type=f32)
    xp += jnp.dot(ent_ref[...] - loop_rel_ref[...], loop_w_ref[...],
                  preferred_element_type=f32)
    xp = xp * (1.0 / 3.0) + bias_ref[...]

    @pl.when(i == 0)
    def _():
      acc[...] = jnp.zeros_like(acc)

    row = i * rblk + lax.broadcasted_iota(jnp.int32, xp.shape, 0)
    xpm = jnp.where(row < n_ent, xp, 0.0)
    acc[0:1, :] += jnp.sum(xpm, axis=0, keepdims=True)
    acc[1:2, :] += jnp.sum(xpm * xpm, axis=0, keepdims=True)

  @pl.when(i == nblk)
  def _():
    xh = jnp.dot(ah_ref[...], in_w_ref[...], preferred_element_type=f32)
    xh += jnp.dot(bh_ref[...], out_w_ref[...], preferred_element_type=f32)
    xh += jnp.dot(ch_ref[...] - loop_rel_ref[...], loop_w_ref[...],
                  preferred_element_type=f32)
    xh = xh * (1.0 / 3.0) + bias_ref[...]
    rh = jnp.dot(rg_ref[...], w_rel_ref[...], preferred_element_type=f32)
    inv_n = 1.0 / n_ent
    mean = acc[0:1, :] * inv_n
    var = acc[1:2, :] * inv_n - mean * mean
    xn = (xh - mean) * lax.rsqrt(var + 1e-5)
    xn = jnp.tanh(xn * gamma_ref[...] + beta_ref[...])
    obj[...] = xn * rh

  @pl.when(i >= nblk)
  def _():
    j = i - nblk
    logits = lax.dot_general(obj[pl.ds(j * bblk, bblk), :], emb_ref[...],
                             (((1,), (1,)), ((), ())),
                             preferred_element_type=jnp.float32)
    logits += bent_ref[...]
    out_ref[...] = jax.nn.sigmoid(logits)


def kernel(ent_emb, rel_emb, in_w, out_w, loop_w, w_rel, loop_rel, bias_cov,
           bn_gamma, bn_beta, b_ent, emb_ent, edge_index, edge_type, edge_norm,
           triples):
  n_ent, d_in = ent_emb.shape
  d_out = in_w.shape[1]
  n_rel = rel_emb.shape[0]
  n_edges = edge_norm.shape[0]
  batch = triples.shape[0]
  chunks = n_edges // _CH
  cpc = chunks // 2
  ns = 16
  cpt = -(-cpc // ns)
  cpt = -(-cpt // 8) * 8
  cpc_pad = cpt * ns

  half = n_edges // 2
  npad = (cpc_pad - cpc) * _CH

  def _chunked(a, fill):
    zpad = jnp.full((npad,), fill, a.dtype)
    return jnp.concatenate([a[:half], zpad, a[half:], zpad])

  src_c = _chunked(edge_index[0], 0)
  dst_c = _chunked(edge_index[1], 0)
  typ_c = _chunked(edge_type, 0)
  nrm_c = _chunked(edge_norm, 0.0)
  n_ent_pad = -(-n_ent // (80 * ns)) * (80 * ns)
  zeros = jnp.zeros((n_ent_pad, d_in), jnp.float32)

  head = jnp.asarray(triples[:, 0], jnp.int32)
  rela = jnp.asarray(triples[:, 1], jnp.int32)
  edge_agg = _make_edge_agg(n_ent_pad, d_in, cpt, batch)
  agg_in, agg_out, ah, bh, ch, rg = edge_agg(
      ent_emb, rel_emb, src_c, typ_c, dst_c, nrm_c, zeros, head, rela)

  ent_p = jnp.concatenate(
      [ent_emb, jnp.zeros((n_ent_pad - n_ent, d_in), jnp.float32)], axis=0)
  rblk = 1024
  nblk = n_ent_pad // rblk
  bblk = 128
  nbb = batch // bblk
  bias2 = bias_cov.reshape(1, d_out)
  last = nblk - 1

  score = pl.pallas_call(
      functools.partial(_fused_body, nblk=nblk, rblk=rblk, n_ent=n_ent,
                        bblk=bblk),
      grid=(nblk + nbb,),
      in_specs=[
          pl.BlockSpec((rblk, d_in), lambda i: (jnp.minimum(i, last), 0)),
          pl.BlockSpec((rblk, d_in), lambda i: (jnp.minimum(i, last), 0)),
          pl.BlockSpec((rblk, d_in), lambda i: (jnp.minimum(i, last), 0)),
          pl.BlockSpec((batch, d_in), lambda i: (0, 0)),
          pl.BlockSpec((batch, d_in), lambda i: (0, 0)),
          pl.BlockSpec((batch, d_in), lambda i: (0, 0)),
          pl.BlockSpec((batch, d_in), lambda i: (0, 0)),
          pl.BlockSpec((d_in, d_out), lambda i: (0, 0)),
          pl.BlockSpec((d_in, d_out), lambda i: (0, 0)),
          pl.BlockSpec((d_in, d_out), lambda i: (0, 0)),
          pl.BlockSpec((d_in, d_out), lambda i: (0, 0)),
          pl.BlockSpec((1, d_in), lambda i: (0, 0)),
          pl.BlockSpec((1, d_out), lambda i: (0, 0)),
          pl.BlockSpec((1, d_out), lambda i: (0, 0)),
          pl.BlockSpec((1, d_out), lambda i: (0, 0)),
          pl.BlockSpec((n_ent, d_out), lambda i: (0, 0)),
          pl.BlockSpec((1, n_ent), lambda i: (0, 0)),
      ],
      out_specs=pl.BlockSpec(
          (bblk, n_ent), lambda i: (jnp.maximum(i - nblk, 0), 0)),
      out_shape=jax.ShapeDtypeStruct((batch, n_ent), jnp.float32),
      scratch_shapes=[pltpu.VMEM((8, d_out), jnp.float32),
                      pltpu.VMEM((batch, d_out), jnp.float32)],
  )(agg_in, agg_out, ent_p, ah, bh, ch, rg, in_w, out_w, loop_w, w_rel,
    loop_rel, bias2, bn_gamma.reshape(1, d_out), bn_beta.reshape(1, d_out),
    emb_ent, b_ent.reshape(1, n_ent))

  return score


# speedup vs baseline: 1.3455x; 1.0018x over previous
import functools

import jax
import jax.numpy as jnp
from jax import lax
from jax.experimental import pallas as pl
from jax.experimental.pallas import tpu as pltpu
from jax.experimental.pallas import tpu_sc as plsc

_CH = 64


def _make_edge_agg(n_ent_pad, d_in, chunks_per_tile, batch):
  mesh = plsc.VectorSubcoreMesh(core_axis_name="c", subcore_axis_name="s")
  ns = 16
  rows_per_tile = n_ent_pad // ns
  nlane = d_in // 16
  nchunks = chunks_per_tile
  hpt = batch // ns

  @functools.partial(
      pl.kernel,
      mesh=mesh,
      out_type=[
          jax.ShapeDtypeStruct((n_ent_pad, d_in), jnp.float32),
          jax.ShapeDtypeStruct((n_ent_pad, d_in), jnp.float32),
          jax.ShapeDtypeStruct((batch, d_in), jnp.float32),
          jax.ShapeDtypeStruct((batch, d_in), jnp.float32),
          jax.ShapeDtypeStruct((batch, d_in), jnp.float32),
          jax.ShapeDtypeStruct((batch, d_in), jnp.float32),
      ],
      scratch_types=[
          pltpu.VMEM_SHARED((n_ent_pad, d_in), jnp.float32),
          pltpu.VMEM((2, _CH), jnp.int32),
          pltpu.VMEM((2, _CH), jnp.int32),
          pltpu.VMEM((3, _CH), jnp.int32),
          pltpu.VMEM((2, _CH), jnp.float32),
          pltpu.VMEM((2, _CH, d_in), jnp.float32),
          pltpu.VMEM((2, _CH, d_in), jnp.float32),
          pltpu.SemaphoreType.DMA,
          pltpu.SemaphoreType.DMA,
          pltpu.SemaphoreType.DMA,
          pltpu.SemaphoreType.DMA,
      ],
  )
  def edge_agg(ent_hbm, rel_hbm, src_hbm, typ_hbm, dst_hbm, nrm_hbm, zero_hbm,
               head_hbm, rela_hbm,
               out_in, out_out, out_ah, out_bh, out_ch, out_rg,
               shared, src_v, typ_v, dst_v, nrm_v,
               erow, rrow, sem_a, sem_b, sem_i, sem_s):
    c = lax.axis_index("c")
    s = lax.axis_index("s")
    rb = s * rows_per_tile
    pltpu.sync_copy(zero_hbm.at[pl.ds(rb, rows_per_tile)],
                    shared.at[pl.ds(rb, rows_per_tile)])
    start = (c * ns + s) * nchunks

    def fire_idx(i, slot, dslot):
      base = (start + i) * _CH
      pltpu.async_copy(src_hbm.at[pl.ds(base, _CH)], src_v.at[slot], sem_i)
      pltpu.async_copy(typ_hbm.at[pl.ds(base, _CH)], typ_v.at[slot], sem_i)
      pltpu.async_copy(dst_hbm.at[pl.ds(base, _CH)], dst_v.at[dslot], sem_i)
      pltpu.async_copy(nrm_hbm.at[pl.ds(base, _CH)], nrm_v.at[slot], sem_i)

    def wait_idx(slot, dslot):
      z = pl.ds(0, _CH)
      pltpu.make_async_copy(src_hbm.at[z], src_v.at[slot], sem_i).wait()
      pltpu.make_async_copy(typ_hbm.at[z], typ_v.at[slot], sem_i).wait()
      pltpu.make_async_copy(dst_hbm.at[z], dst_v.at[dslot], sem_i).wait()
      pltpu.make_async_copy(nrm_hbm.at[z], nrm_v.at[slot], sem_i).wait()

    def wait_scatter(dslot):
      pltpu.make_async_copy(erow.at[0], shared.at[dst_v.at[dslot]],
                            sem_s).wait()

    def fire_rows(slot):
      pltpu.async_copy(ent_hbm.at[src_v.at[slot]], erow.at[slot], sem_a)
      pltpu.async_copy(rel_hbm.at[typ_v.at[slot]], rrow.at[slot], sem_b)

    def drain_rows(slot):
      pltpu.make_async_copy(ent_hbm.at[src_v.at[0]], erow.at[slot], sem_a).wait()
      pltpu.make_async_copy(rel_hbm.at[typ_v.at[0]], rrow.at[slot], sem_b).wait()

    fire_idx(0, 0, 0)
    wait_idx(0, 0)
    fire_rows(0)
    fire_idx(1, 1, 1)
    plsc.subcore_barrier()

    dnums = lax.GatherDimensionNumbers(
        offset_dims=(), collapsed_slice_dims=(0,), start_index_map=(0,))

    def chunk_body(i, carry):
      slot = lax.rem(i, 2)
      nxt = lax.rem(i + 1, 2)
      dslot = lax.rem(i, 3)

      drain_rows(slot)

      @pl.when(i >= 1)
      def _():
        wait_scatter(lax.rem(i + 2, 3))

      @pl.when(i + 1 < nchunks)
      def _():
        wait_idx(nxt, lax.rem(i + 1, 3))
        fire_rows(nxt)

      def group_body(g, carry2):
        gv = nrm_v[slot, pl.ds(g * 16, 16)]
        for lane in range(16):
          lidx = jnp.full((16, 1), lane, jnp.int32)
          n = lax.gather(gv, lidx, dnums, slice_sizes=(1,),
                         mode=lax.GatherScatterMode.PROMISE_IN_BOUNDS)
          e = g * 16 + lane
          for j in range(nlane):
            sl = pl.ds(j * 16, 16)
            erow[slot, e, sl] = (erow[slot, e, sl] - rrow[slot, e, sl]) * n
        return carry2

      lax.fori_loop(0, _CH // 16, group_body, 0)
      pltpu.async_copy(erow.at[slot], shared.at[dst_v.at[dslot]], sem_s,
                       add=True)

      @pl.when(i + 2 < nchunks)
      def _():
        fire_idx(i + 2, slot, lax.rem(i + 2, 3))

      return carry

    lax.fori_loop(0, nchunks, chunk_body, 0)
    wait_scatter(lax.rem(nchunks - 1, 3))
    plsc.subcore_barrier()

    gb = s * hpt

    @pl.when(c == 0)
    def _():
      pltpu.sync_copy(shared.at[pl.ds(rb, rows_per_tile)],
                      out_in.at[pl.ds(rb, rows_per_tile)])
      pltpu.sync_copy(head_hbm.at[pl.ds(gb, hpt)], src_v.at[0])
      pltpu.sync_copy(shared.at[src_v.at[0]], erow.at[0])
      pltpu.sync_copy(ent_hbm.at[src_v.at[0]], rrow.at[0])
      pltpu.sync_copy(erow.at[0], out_ah.at[pl.ds(gb, hpt)])
      pltpu.sync_copy(rrow.at[0], out_ch.at[pl.ds(gb, hpt)])

    @pl.when(c == 1)
    def _():
      pltpu.sync_copy(shared.at[pl.ds(rb, rows_per_tile)],
                      out_out.at[pl.ds(rb, rows_per_tile)])
      pltpu.sync_copy(head_hbm.at[pl.ds(gb, hpt)], src_v.at[0])
      pltpu.sync_copy(rela_hbm.at[pl.ds(gb, hpt)], typ_v.at[0])
      pltpu.sync_copy(shared.at[src_v.at[0]], erow.at[0])
      pltpu.sync_copy(rel_hbm.at[typ_v.at[0]], rrow.at[0])
      pltpu.sync_copy(erow.at[0], out_bh.at[pl.ds(gb, hpt)])
      pltpu.sync_copy(rrow.at[0], out_rg.at[pl.ds(gb, hpt)])

  return edge_agg


def _fused_body(agg_in_ref, agg_out_ref, ent_ref, ah_ref, bh_ref, ch_ref,
                rg_ref, in_w_ref, out_w_ref, loop_w_ref, w_rel_ref,
                loop_rel_ref, bias_ref, gamma_ref, beta_ref, emb_ref,
                bent_ref, out_ref, acc, obj, *, nblk, rblk, n_ent, bblk):
  i = pl.program_id(0)
  f32 = jnp.float32

  @pl.when(i < nblk)
  def _():
    xp = jnp.dot(agg_in_ref[...], in_w_ref[...], preferred_element_type=f32)
    xp += jnp.dot(agg_out_ref[...], out_w_ref[...], preferred_element_type=f32)
    xp += jnp.dot(ent_ref[...] - loop_rel_ref[...], loop_w_ref[...],
                  preferred_element_type=f32)
    xp = xp * (1.0 / 3.0) + bias_ref[...]

    @pl.when(i == 0)
    def _():
      acc[...] = jnp.zeros_like(acc)

    row = i * rblk + lax.broadcasted_iota(jnp.int32, xp.shape, 0)
    xpm = jnp.where(row < n_ent, xp, 0.0)
    acc[0:1, :] += jnp.sum(xpm, axis=0, keepdims=True)
    acc[1:2, :] += jnp.sum(xpm * xpm, axis=0, keepdims=True)

  @pl.when(i == nblk)
  def _():
    xh = jnp.dot(ah_ref[...], in_w_ref[...], preferred_element_type=f32)
    xh += jnp.dot(bh_ref[...], out_w_ref[...], preferred_element_type=f32)
    xh += jnp.dot(ch_ref[...] - loop_rel_ref[...], loop_w_ref[...],
                  preferred_element_type=f32)
    xh = xh * (1.0 / 3.0) + bias_ref[...]
    rh = jnp.dot(rg_ref[...], w_rel_ref[...], preferred_element_type=f32)
    inv_n = 1.0 / n_ent
    mean = acc[0:1, :] * inv_n
    var = acc[1:2, :] * inv_n - mean * mean
    xn = (xh - mean) * lax.rsqrt(var + 1e-5)
    xn = jnp.tanh(xn * gamma_ref[...] + beta_ref[...])
    obj[...] = xn * rh

  @pl.when(i >= nblk)
  def _():
    j = i - nblk
    logits = lax.dot_general(obj[pl.ds(j * bblk, bblk), :], emb_ref[...],
                             (((1,), (1,)), ((), ())),
                             preferred_element_type=jnp.float32)
    logits += bent_ref[...]
    out_ref[...] = jax.nn.sigmoid(logits)


def kernel(ent_emb, rel_emb, in_w, out_w, loop_w, w_rel, loop_rel, bias_cov,
           bn_gamma, bn_beta, b_ent, emb_ent, edge_index, edge_type, edge_norm,
           triples):
  n_ent, d_in = ent_emb.shape
  d_out = in_w.shape[1]
  n_rel = rel_emb.shape[0]
  n_edges = edge_norm.shape[0]
  batch = triples.shape[0]
  chunks = n_edges // _CH
  cpc = chunks // 2
  ns = 16
  cpt = -(-cpc // ns)
  cpt = -(-cpt // 8) * 8
  cpc_pad = cpt * ns

  half = n_edges // 2
  npad = (cpc_pad - cpc) * _CH

  def _chunked(a, fill):
    zpad = jnp.full((npad,), fill, a.dtype)
    return jnp.concatenate([a[:half], zpad, a[half:], zpad])

  src_c = _chunked(edge_index[0], 0)
  dst_c = _chunked(edge_index[1], 0)
  typ_c = _chunked(edge_type, 0)
  nrm_c = _chunked(edge_norm, 0.0)
  n_ent_pad = -(-n_ent // (80 * ns)) * (80 * ns)
  zeros = jnp.zeros((n_ent_pad, d_in), jnp.float32)

  head = jnp.asarray(triples[:, 0], jnp.int32)
  rela = jnp.asarray(triples[:, 1], jnp.int32)
  edge_agg = _make_edge_agg(n_ent_pad, d_in, cpt, batch)
  agg_in, agg_out, ah, bh, ch, rg = edge_agg(
      ent_emb, rel_emb, src_c, typ_c, dst_c, nrm_c, zeros, head, rela)

  ent_p = jnp.concatenate(
      [ent_emb, jnp.zeros((n_ent_pad - n_ent, d_in), jnp.float32)], axis=0)
  rblk = 1024
  nblk = n_ent_pad // rblk
  bblk = 128
  nbb = batch // bblk
  bias2 = bias_cov.reshape(1, d_out)
  last = nblk - 1

  score = pl.pallas_call(
      functools.partial(_fused_body, nblk=nblk, rblk=rblk, n_ent=n_ent,
                        bblk=bblk),
      grid=(nblk + nbb,),
      in_specs=[
          pl.BlockSpec((rblk, d_in), lambda i: (jnp.minimum(i, last), 0)),
          pl.BlockSpec((rblk, d_in), lambda i: (jnp.minimum(i, last), 0)),
          pl.BlockSpec((rblk, d_in), lambda i: (jnp.minimum(i, last), 0)),
          pl.BlockSpec((batch, d_in), lambda i: (0, 0)),
          pl.BlockSpec((batch, d_in), lambda i: (0, 0)),
          pl.BlockSpec((batch, d_in), lambda i: (0, 0)),
          pl.BlockSpec((batch, d_in), lambda i: (0, 0)),
          pl.BlockSpec((d_in, d_out), lambda i: (0, 0)),
          pl.BlockSpec((d_in, d_out), lambda i: (0, 0)),
          pl.BlockSpec((d_in, d_out), lambda i: (0, 0)),
          pl.BlockSpec((d_in, d_out), lambda i: (0, 0)),
          pl.BlockSpec((1, d_in), lambda i: (0, 0)),
          pl.BlockSpec((1, d_out), lambda i: (0, 0)),
          pl.BlockSpec((1, d_out), lambda i: (0, 0)),
          pl.BlockSpec((1, d_out), lambda i: (0, 0)),
          pl.BlockSpec((n_ent, d_out), lambda i: (0, 0)),
          pl.BlockSpec((1, n_ent), lambda i: (0, 0)),
      ],
      out_specs=pl.BlockSpec(
          (bblk, n_ent), lambda i: (jnp.maximum(i - nblk, 0), 0)),
      out_shape=jax.ShapeDtypeStruct((batch, n_ent), jnp.float32),
      scratch_shapes=[pltpu.VMEM((8, d_out), jnp.float32),
                      pltpu.VMEM((batch, d_out), jnp.float32)],
  )(agg_in, agg_out, ent_p, ah, bh, ch, rg, in_w, out_w, loop_w, w_rel,
    loop_rel, bias2, bn_gamma.reshape(1, d_out), bn_beta.reshape(1, d_out),
    emb_ent, b_ent.reshape(1, n_ent))

  return score


# speedup vs baseline: 1.3488x; 1.0025x over previous
import functools

import jax
import jax.numpy as jnp
from jax import lax
from jax.experimental import pallas as pl
from jax.experimental.pallas import tpu as pltpu
from jax.experimental.pallas import tpu_sc as plsc

_CH = 64


def _make_edge_agg(n_ent_pad, d_in, chunks_per_tile, batch):
  mesh = plsc.VectorSubcoreMesh(core_axis_name="c", subcore_axis_name="s")
  ns = 16
  rows_per_tile = n_ent_pad // ns
  nlane = d_in // 16
  nchunks = chunks_per_tile
  hpt = batch // ns

  @functools.partial(
      pl.kernel,
      mesh=mesh,
      out_type=[
          jax.ShapeDtypeStruct((n_ent_pad, d_in), jnp.float32),
          jax.ShapeDtypeStruct((n_ent_pad, d_in), jnp.float32),
          jax.ShapeDtypeStruct((batch, d_in), jnp.float32),
          jax.ShapeDtypeStruct((batch, d_in), jnp.float32),
          jax.ShapeDtypeStruct((batch, d_in), jnp.float32),
          jax.ShapeDtypeStruct((batch, d_in), jnp.float32),
      ],
      scratch_types=[
          pltpu.VMEM_SHARED((n_ent_pad, d_in), jnp.float32),
          pltpu.VMEM((2, _CH), jnp.int32),
          pltpu.VMEM((2, _CH), jnp.int32),
          pltpu.VMEM((3, _CH), jnp.int32),
          pltpu.VMEM((nchunks * _CH,), jnp.float32),
          pltpu.VMEM((2, _CH, d_in), jnp.float32),
          pltpu.VMEM((2, _CH, d_in), jnp.float32),
          pltpu.SemaphoreType.DMA,
          pltpu.SemaphoreType.DMA,
          pltpu.SemaphoreType.DMA,
          pltpu.SemaphoreType.DMA,
      ],
  )
  def edge_agg(ent_hbm, rel_hbm, src_hbm, typ_hbm, dst_hbm, nrm_hbm, zero_hbm,
               head_hbm, rela_hbm,
               out_in, out_out, out_ah, out_bh, out_ch, out_rg,
               shared, src_v, typ_v, dst_v, nrm_v,
               erow, rrow, sem_a, sem_b, sem_i, sem_s):
    c = lax.axis_index("c")
    s = lax.axis_index("s")
    rb = s * rows_per_tile
    pltpu.sync_copy(zero_hbm.at[pl.ds(rb, rows_per_tile)],
                    shared.at[pl.ds(rb, rows_per_tile)])
    start = (c * ns + s) * nchunks

    pltpu.async_copy(nrm_hbm.at[pl.ds(start * _CH, nchunks * _CH)], nrm_v,
                     sem_s)

    def fire_idx(i, slot, dslot):
      base = (start + i) * _CH
      pltpu.async_copy(src_hbm.at[pl.ds(base, _CH)], src_v.at[slot], sem_i)
      pltpu.async_copy(typ_hbm.at[pl.ds(base, _CH)], typ_v.at[slot], sem_i)
      pltpu.async_copy(dst_hbm.at[pl.ds(base, _CH)], dst_v.at[dslot], sem_i)

    def wait_idx(slot, dslot):
      z = pl.ds(0, _CH)
      pltpu.make_async_copy(src_hbm.at[z], src_v.at[slot], sem_i).wait()
      pltpu.make_async_copy(typ_hbm.at[z], typ_v.at[slot], sem_i).wait()
      pltpu.make_async_copy(dst_hbm.at[z], dst_v.at[dslot], sem_i).wait()

    def wait_scatter(dslot):
      pltpu.make_async_copy(erow.at[0], shared.at[dst_v.at[dslot]],
                            sem_s).wait()

    def fire_rows(slot):
      pltpu.async_copy(ent_hbm.at[src_v.at[slot]], erow.at[slot], sem_a)
      pltpu.async_copy(rel_hbm.at[typ_v.at[slot]], rrow.at[slot], sem_b)

    def drain_rows(slot):
      pltpu.make_async_copy(ent_hbm.at[src_v.at[0]], erow.at[slot], sem_a).wait()
      pltpu.make_async_copy(rel_hbm.at[typ_v.at[0]], rrow.at[slot], sem_b).wait()

    fire_idx(0, 0, 0)
    wait_idx(0, 0)
    fire_rows(0)
    fire_idx(1, 1, 1)
    pltpu.make_async_copy(nrm_hbm.at[pl.ds(0, nchunks * _CH)], nrm_v,
                          sem_s).wait()
    plsc.subcore_barrier()

    dnums = lax.GatherDimensionNumbers(
        offset_dims=(), collapsed_slice_dims=(0,), start_index_map=(0,))

    def chunk_body(i, carry):
      slot = lax.rem(i, 2)
      nxt = lax.rem(i + 1, 2)
      dslot = lax.rem(i, 3)

      drain_rows(slot)

      @pl.when(i >= 1)
      def _():
        wait_scatter(lax.rem(i + 2, 3))

      @pl.when(i + 1 < nchunks)
      def _():
        wait_idx(nxt, lax.rem(i + 1, 3))
        fire_rows(nxt)

      def group_body(g, carry2):
        gv = nrm_v[pl.ds(i * _CH + g * 16, 16)]
        for lane in range(16):
          lidx = jnp.full((16, 1), lane, jnp.int32)
          n = lax.gather(gv, lidx, dnums, slice_sizes=(1,),
                         mode=lax.GatherScatterMode.PROMISE_IN_BOUNDS)
          e = g * 16 + lane
          for j in range(nlane):
            sl = pl.ds(j * 16, 16)
            erow[slot, e, sl] = (erow[slot, e, sl] - rrow[slot, e, sl]) * n
        return carry2

      lax.fori_loop(0, _CH // 16, group_body, 0)
      pltpu.async_copy(erow.at[slot], shared.at[dst_v.at[dslot]], sem_s,
                       add=True)

      @pl.when(i + 2 < nchunks)
      def _():
        fire_idx(i + 2, slot, lax.rem(i + 2, 3))

      return carry

    lax.fori_loop(0, nchunks, chunk_body, 0)
    wait_scatter(lax.rem(nchunks - 1, 3))
    plsc.subcore_barrier()

    gb = s * hpt

    @pl.when(c == 0)
    def _():
      pltpu.sync_copy(shared.at[pl.ds(rb, rows_per_tile)],
                      out_in.at[pl.ds(rb, rows_per_tile)])
      pltpu.sync_copy(head_hbm.at[pl.ds(gb, hpt)], src_v.at[0])
      pltpu.sync_copy(shared.at[src_v.at[0]], erow.at[0])
      pltpu.sync_copy(ent_hbm.at[src_v.at[0]], rrow.at[0])
      pltpu.sync_copy(erow.at[0], out_ah.at[pl.ds(gb, hpt)])
      pltpu.sync_copy(rrow.at[0], out_ch.at[pl.ds(gb, hpt)])

    @pl.when(c == 1)
    def _():
      pltpu.sync_copy(shared.at[pl.ds(rb, rows_per_tile)],
                      out_out.at[pl.ds(rb, rows_per_tile)])
      pltpu.sync_copy(head_hbm.at[pl.ds(gb, hpt)], src_v.at[0])
      pltpu.sync_copy(rela_hbm.at[pl.ds(gb, hpt)], typ_v.at[0])
      pltpu.sync_copy(shared.at[src_v.at[0]], erow.at[0])
      pltpu.sync_copy(rel_hbm.at[typ_v.at[0]], rrow.at[0])
      pltpu.sync_copy(erow.at[0], out_bh.at[pl.ds(gb, hpt)])
      pltpu.sync_copy(rrow.at[0], out_rg.at[pl.ds(gb, hpt)])

  return edge_agg


def _fused_body(agg_in_ref, agg_out_ref, ent_ref, ah_ref, bh_ref, ch_ref,
                rg_ref, in_w_ref, out_w_ref, loop_w_ref, w_rel_ref,
                loop_rel_ref, bias_ref, gamma_ref, beta_ref, emb_ref,
                bent_ref, out_ref, acc, obj, *, nblk, rblk, n_ent, bblk):
  i = pl.program_id(0)
  f32 = jnp.float32

  @pl.when(i < nblk)
  def _():
    xp = jnp.dot(agg_in_ref[...], in_w_ref[...], preferred_element_type=f32)
    xp += jnp.dot(agg_out_ref[...], out_w_ref[...], preferred_element_type=f32)
    xp += jnp.dot(ent_ref[...] - loop_rel_ref[...], loop_w_ref[...],
                  preferred_element_type=f32)
    xp = xp * (1.0 / 3.0) + bias_ref[...]

    @pl.when(i == 0)
    def _():
      acc[...] = jnp.zeros_like(acc)

    row = i * rblk + lax.broadcasted_iota(jnp.int32, xp.shape, 0)
    xpm = jnp.where(row < n_ent, xp, 0.0)
    acc[0:1, :] += jnp.sum(xpm, axis=0, keepdims=True)
    acc[1:2, :] += jnp.sum(xpm * xpm, axis=0, keepdims=True)

  @pl.when(i == nblk)
  def _():
    xh = jnp.dot(ah_ref[...], in_w_ref[...], preferred_element_type=f32)
    xh += jnp.dot(bh_ref[...], out_w_ref[...], preferred_element_type=f32)
    xh += jnp.dot(ch_ref[...] - loop_rel_ref[...], loop_w_ref[...],
                  preferred_element_type=f32)
    xh = xh * (1.0 / 3.0) + bias_ref[...]
    rh = jnp.dot(rg_ref[...], w_rel_ref[...], preferred_element_type=f32)
    inv_n = 1.0 / n_ent
    mean = acc[0:1, :] * inv_n
    var = acc[1:2, :] * inv_n - mean * mean
    xn = (xh - mean) * lax.rsqrt(var + 1e-5)
    xn = jnp.tanh(xn * gamma_ref[...] + beta_ref[...])
    obj[...] = xn * rh

  @pl.when(i >= nblk)
  def _():
    j = i - nblk
    logits = lax.dot_general(obj[pl.ds(j * bblk, bblk), :], emb_ref[...],
                             (((1,), (1,)), ((), ())),
                             preferred_element_type=jnp.float32)
    logits += bent_ref[...]
    out_ref[...] = jax.nn.sigmoid(logits)


def kernel(ent_emb, rel_emb, in_w, out_w, loop_w, w_rel, loop_rel, bias_cov,
           bn_gamma, bn_beta, b_ent, emb_ent, edge_index, edge_type, edge_norm,
           triples):
  n_ent, d_in = ent_emb.shape
  d_out = in_w.shape[1]
  n_rel = rel_emb.shape[0]
  n_edges = edge_norm.shape[0]
  batch = triples.shape[0]
  chunks = n_edges // _CH
  cpc = chunks // 2
  ns = 16
  cpt = -(-cpc // ns)
  cpt = -(-cpt // 8) * 8
  cpc_pad = cpt * ns

  half = n_edges // 2
  npad = (cpc_pad - cpc) * _CH

  def _chunked(a, fill):
    zpad = jnp.full((npad,), fill, a.dtype)
    return jnp.concatenate([a[:half], zpad, a[half:], zpad])

  src_c = _chunked(edge_index[0], 0)
  dst_c = _chunked(edge_index[1], 0)
  typ_c = _chunked(edge_type, 0)
  nrm_c = _chunked(edge_norm, 0.0)
  n_ent_pad = -(-n_ent // (80 * ns)) * (80 * ns)
  zeros = jnp.zeros((n_ent_pad, d_in), jnp.float32)

  head = jnp.asarray(triples[:, 0], jnp.int32)
  rela = jnp.asarray(triples[:, 1], jnp.int32)
  edge_agg = _make_edge_agg(n_ent_pad, d_in, cpt, batch)
  agg_in, agg_out, ah, bh, ch, rg = edge_agg(
      ent_emb, rel_emb, src_c, typ_c, dst_c, nrm_c, zeros, head, rela)

  ent_p = jnp.concatenate(
      [ent_emb, jnp.zeros((n_ent_pad - n_ent, d_in), jnp.float32)], axis=0)
  rblk = 1024
  nblk = n_ent_pad // rblk
  bblk = 128
  nbb = batch // bblk
  bias2 = bias_cov.reshape(1, d_out)
  last = nblk - 1

  score = pl.pallas_call(
      functools.partial(_fused_body, nblk=nblk, rblk=rblk, n_ent=n_ent,
                        bblk=bblk),
      grid=(nblk + nbb,),
      in_specs=[
          pl.BlockSpec((rblk, d_in), lambda i: (jnp.minimum(i, last), 0)),
          pl.BlockSpec((rblk, d_in), lambda i: (jnp.minimum(i, last), 0)),
          pl.BlockSpec((rblk, d_in), lambda i: (jnp.minimum(i, last), 0)),
          pl.BlockSpec((batch, d_in), lambda i: (0, 0)),
          pl.BlockSpec((batch, d_in), lambda i: (0, 0)),
          pl.BlockSpec((batch, d_in), lambda i: (0, 0)),
          pl.BlockSpec((batch, d_in), lambda i: (0, 0)),
          pl.BlockSpec((d_in, d_out), lambda i: (0, 0)),
          pl.BlockSpec((d_in, d_out), lambda i: (0, 0)),
          pl.BlockSpec((d_in, d_out), lambda i: (0, 0)),
          pl.BlockSpec((d_in, d_out), lambda i: (0, 0)),
          pl.BlockSpec((1, d_in), lambda i: (0, 0)),
          pl.BlockSpec((1, d_out), lambda i: (0, 0)),
          pl.BlockSpec((1, d_out), lambda i: (0, 0)),
          pl.BlockSpec((1, d_out), lambda i: (0, 0)),
          pl.BlockSpec((n_ent, d_out), lambda i: (0, 0)),
          pl.BlockSpec((1, n_ent), lambda i: (0, 0)),
      ],
      out_specs=pl.BlockSpec(
          (bblk, n_ent), lambda i: (jnp.maximum(i - nblk, 0), 0)),
      out_shape=jax.ShapeDtypeStruct((batch, n_ent), jnp.float32),
      scratch_shapes=[pltpu.VMEM((8, d_out), jnp.float32),
                      pltpu.VMEM((batch, d_out), jnp.float32)],
  )(agg_in, agg_out, ent_p, ah, bh, ch, rg, in_w, out_w, loop_w, w_rel,
    loop_rel, bias2, bn_gamma.reshape(1, d_out), bn_beta.reshape(1, d_out),
    emb_ent, b_ent.reshape(1, n_ent))

  return score
